# Initial kernel scaffold; baseline (speedup 1.0000x reference)
import jax
import jax.numpy as jnp
from jax.experimental import pallas as pl


def kernel(x, edge_index, W1, b1, W2, b2, Wfc, bfc):
    raise NotImplementedError("write your pallas kernel here")


# speedup vs baseline: 7.9272x; 7.9272x over previous
import functools

import jax
import jax.numpy as jnp
from jax import lax
from jax.experimental import pallas as pl
from jax.experimental.pallas import tpu as pltpu
from jax.experimental.pallas import tpu_sc as plsc

N = 10000
NPAD = 10240
E = 320000
LANES = 128
CH = 158
EPT = CH * LANES
EPAD = 16 * EPT
CHD = CH // 2
D = 128
DEGW = 128
RPT = NPAD // 16
BLK = 256

@functools.cache
def _sc_kernels():
    mesh = plsc.VectorSubcoreMesh(
        core_axis_name="c", subcore_axis_name="s", num_cores=2, num_subcores=16
    )

    @functools.partial(
        pl.kernel,
        out_type=jax.ShapeDtypeStruct((2 * NPAD, DEGW), jnp.float32),
        mesh=mesh,
        scratch_types=[
            pltpu.VMEM((LANES,), jnp.int32),
            pltpu.VMEM((LANES, DEGW), jnp.float32),
            pltpu.VMEM_SHARED((NPAD, DEGW), jnp.float32),
        ],
    )
    def sc_degree(dst_hbm, out_hbm, dbuf, ones_v, acc):
        c = lax.axis_index("c")
        s = lax.axis_index("s")

        def fill_zeros(i, _):
            ones_v[i // 8, pl.ds((i % 8) * 16, 16)] = jnp.zeros(
                (16,), jnp.float32
            )
            return 0

        lax.fori_loop(0, LANES * 8, fill_zeros, 0)
        for k in range(RPT // LANES):
            pltpu.sync_copy(ones_v, acc.at[pl.ds(s * RPT + k * LANES, LANES)])
        plsc.subcore_barrier()

        def fill_ones(i, _):
            ones_v[i // 8, pl.ds((i % 8) * 16, 16)] = jnp.full(
                (16,), 1.0, jnp.float32
            )
            return 0

        lax.fori_loop(0, LANES * 8, fill_ones, 0)

        def step(g, _):
            pltpu.sync_copy(dst_hbm.at[s * CH + c * CHD + g], dbuf)
            pltpu.sync_copy(ones_v, acc.at[dbuf], add=True)
            return 0

        lax.fori_loop(0, CHD, step, 0)
        plsc.subcore_barrier()
        for k in range(RPT // LANES):
            pltpu.sync_copy(
                acc.at[pl.ds(s * RPT + k * LANES, LANES)],
                out_hbm.at[pl.ds(c * NPAD + s * RPT + k * LANES, LANES)],
            )

    @functools.partial(
        pl.kernel,
        out_type=jax.ShapeDtypeStruct((2 * NPAD, D), jnp.float32),
        mesh=mesh,
        scratch_types=[
            pltpu.VMEM((LANES,), jnp.int32),
            pltpu.VMEM((LANES,), jnp.int32),
            pltpu.VMEM((LANES, D), jnp.float32),
            pltpu.VMEM_SHARED((NPAD, D), jnp.float32),
            pltpu.SemaphoreType.DMA,
        ],
    )
    def sc_aggregate(table_hbm, src_hbm, dst_hbm, out_hbm,
                     sbuf, dbuf, rbuf, acc, gsem):
        c = lax.axis_index("c")
        s = lax.axis_index("s")
        w = c * 16 + s

        def fz(i, _):
            rbuf[i // 8, pl.ds((i % 8) * 16, 16)] = jnp.zeros((16,), jnp.float32)
            return 0

        lax.fori_loop(0, LANES * 8, fz, 0)
        for k in range(RPT // LANES):
            pltpu.sync_copy(rbuf, acc.at[pl.ds(s * RPT + k * LANES, LANES)])
        plsc.subcore_barrier()

        def step(g, _):
            pltpu.sync_copy(src_hbm.at[w * CH + g], sbuf)
            pltpu.sync_copy(dst_hbm.at[s * CH + g], dbuf)
            pltpu.async_copy(table_hbm.at[sbuf], rbuf, gsem).wait()
            pltpu.sync_copy(rbuf, acc.at[dbuf], add=True)
            return 0

        lax.fori_loop(0, CH, step, 0)
        plsc.subcore_barrier()
        for k in range(RPT // LANES):
            pltpu.sync_copy(
                acc.at[pl.ds(s * RPT + k * LANES, LANES)],
                out_hbm.at[pl.ds(c * NPAD + s * RPT + k * LANES, LANES)],
            )

    return sc_degree, sc_aggregate


def _tc1_body(x_ref, w_ref, deg_ref, out_ref):
    xw = jnp.dot(x_ref[...], w_ref[...], preferred_element_type=jnp.float32)
    deg = deg_ref[0] + deg_ref[1] + 1.0
    dinv = lax.rsqrt(deg)[:, 0:1]
    xs = xw * dinv
    out_ref[0] = xs[:, :D]
    out_ref[1] = xs[:, D:]


def _tc2_body(agg_ref, xs_ref, deg_ref, w_ref, b_ref, out_ref):
    a = jnp.concatenate(
        [agg_ref[0] + xs_ref[0], agg_ref[1] + xs_ref[1]], axis=1
    )
    deg = deg_ref[0] + deg_ref[1] + 1.0
    dinv = lax.rsqrt(deg)[:, 0:1]
    h = jnp.maximum(a * dinv + b_ref[...], 0.0)
    hw = jnp.dot(h, w_ref[...], preferred_element_type=jnp.float32)
    hs = hw * dinv
    out_ref[0] = hs[:, :D]
    out_ref[1] = hs[:, D:]


def _tc3_body(agg_ref, xs_ref, deg_ref, b_ref, wfc_ref, bfc_ref, out_ref):
    a = jnp.concatenate(
        [agg_ref[0] + xs_ref[0], agg_ref[1] + xs_ref[1]], axis=1
    )
    deg = deg_ref[0] + deg_ref[1] + 1.0
    dinv = lax.rsqrt(deg)[:, 0:1]
    h = jnp.maximum(a * dinv + b_ref[...], 0.0)
    out_ref[...] = (
        jnp.dot(h, wfc_ref[...], preferred_element_type=jnp.float32)
        + bfc_ref[...]
    )


_GRID = (NPAD // BLK,)


def _row_spec(shape):
    nd = len(shape)
    if nd == 2:
        return pl.BlockSpec((BLK, shape[1]), lambda i: (i, 0))
    return pl.BlockSpec((shape[0], BLK, shape[2]), lambda i: (0, i, 0))


def _full_spec(shape):
    return pl.BlockSpec(shape, lambda i: (0,) * len(shape))


def _tc1(xpad, W1, deg2):
    return pl.pallas_call(
        _tc1_body,
        grid=_GRID,
        in_specs=[
            _row_spec((NPAD, 128)),
            _full_spec((128, 256)),
            _row_spec((2, NPAD, DEGW)),
        ],
        out_specs=_row_spec((2, NPAD, D)),
        out_shape=jax.ShapeDtypeStruct((2, NPAD, D), jnp.float32),
    )(xpad, W1, deg2)


def _tc2(agg1, xw1s, deg2, W2, b1):
    return pl.pallas_call(
        _tc2_body,
        grid=_GRID,
        in_specs=[
            _row_spec((2, NPAD, D)),
            _row_spec((2, NPAD, D)),
            _row_spec((2, NPAD, DEGW)),
            _full_spec((256, 256)),
            _full_spec((1, 256)),
        ],
        out_specs=_row_spec((2, NPAD, D)),
        out_shape=jax.ShapeDtypeStruct((2, NPAD, D), jnp.float32),
    )(agg1, xw1s, deg2, W2, b1)


def _tc3(agg2, h1s, deg2, b2, Wfc, bfc):
    return pl.pallas_call(
        _tc3_body,
        grid=_GRID,
        in_specs=[
            _row_spec((2, NPAD, D)),
            _row_spec((2, NPAD, D)),
            _row_spec((2, NPAD, DEGW)),
            _full_spec((1, 256)),
            _full_spec((256, 128)),
            _full_spec((1, 128)),
        ],
        out_specs=_row_spec((NPAD, 128)),
        out_shape=jax.ShapeDtypeStruct((NPAD, 128), jnp.float32),
    )(agg2, h1s, deg2, b2, Wfc, bfc)


def kernel(x, edge_index, W1, b1, W2, b2, Wfc, bfc):
    src = edge_index[0].astype(jnp.int32)
    dst = edge_index[1].astype(jnp.int32)
    pad = EPAD - E
    srcp = jnp.concatenate([src, jnp.zeros((pad,), jnp.int32)])
    dstp = jnp.concatenate([dst, jnp.full((pad,), N, jnp.int32)])
    src2 = jnp.concatenate([srcp, srcp + NPAD]).reshape(32 * CH, LANES)
    dstf = dstp.reshape(16 * CH, LANES)
    xpad = jnp.pad(x, ((0, NPAD - N), (0, 0)))

    sc_degree, sc_aggregate = _sc_kernels()
    deg2 = sc_degree(dstf).reshape(2, NPAD, DEGW)
    xw1s = _tc1(xpad, W1, deg2)
    agg1 = sc_aggregate(xw1s.reshape(2 * NPAD, D), src2, dstf)
    h1s = _tc2(agg1.reshape(2, NPAD, D), xw1s, deg2, W2, b1.reshape(1, 256))
    agg2 = sc_aggregate(h1s.reshape(2 * NPAD, D), src2, dstf)
    out = _tc3(
        agg2.reshape(2, NPAD, D),
        h1s,
        deg2,
        b2.reshape(1, 256),
        Wfc,
        bfc.reshape(1, 128),
    )
    return out[:N]


# speedup vs baseline: 8.1279x; 1.0253x over previous
import functools

import jax
import jax.numpy as jnp
from jax import lax
from jax.experimental import pallas as pl
from jax.experimental.pallas import tpu as pltpu
from jax.experimental.pallas import tpu_sc as plsc

N = 10000
NPAD = 10240
E = 320000
LANES = 128
CH = 160
GRP = 8
EPT = CH * LANES
EPAD = 16 * EPT
CHD = CH // 2
D = 128
DEGW = 128
RPT = NPAD // 16
BLK = 256

@functools.cache
def _sc_kernels():
    mesh = plsc.VectorSubcoreMesh(
        core_axis_name="c", subcore_axis_name="s", num_cores=2, num_subcores=16
    )

    @functools.partial(
        pl.kernel,
        out_type=jax.ShapeDtypeStruct((2 * NPAD, DEGW), jnp.float32),
        mesh=mesh,
        scratch_types=[
            pltpu.VMEM((LANES,), jnp.int32),
            pltpu.VMEM((LANES, DEGW), jnp.float32),
            pltpu.VMEM_SHARED((NPAD, DEGW), jnp.float32),
        ],
    )
    def sc_degree(dst_hbm, out_hbm, dbuf, ones_v, acc):
        c = lax.axis_index("c")
        s = lax.axis_index("s")

        def fill_zeros(i, _):
            ones_v[i // 8, pl.ds((i % 8) * 16, 16)] = jnp.zeros(
                (16,), jnp.float32
            )
            return 0

        lax.fori_loop(0, LANES * 8, fill_zeros, 0)
        for k in range(RPT // LANES):
            pltpu.sync_copy(ones_v, acc.at[pl.ds(s * RPT + k * LANES, LANES)])
        plsc.subcore_barrier()

        def fill_ones(i, _):
            ones_v[i // 8, pl.ds((i % 8) * 16, 16)] = jnp.full(
                (16,), 1.0, jnp.float32
            )
            return 0

        lax.fori_loop(0, LANES * 8, fill_ones, 0)

        def step(g, _):
            pltpu.sync_copy(dst_hbm.at[s * CH + c * CHD + g], dbuf)
            pltpu.sync_copy(ones_v, acc.at[dbuf], add=True)
            return 0

        lax.fori_loop(0, CHD, step, 0)
        plsc.subcore_barrier()
        for k in range(RPT // LANES):
            pltpu.sync_copy(
                acc.at[pl.ds(s * RPT + k * LANES, LANES)],
                out_hbm.at[pl.ds(c * NPAD + s * RPT + k * LANES, LANES)],
            )

    @functools.partial(
        pl.kernel,
        out_type=jax.ShapeDtypeStruct((2 * NPAD, D), jnp.float32),
        mesh=mesh,
        scratch_types=[
            pltpu.VMEM((GRP, LANES), jnp.int32),
            pltpu.VMEM((GRP, LANES), jnp.int32),
            pltpu.VMEM((LANES, D), jnp.float32),
            pltpu.VMEM((LANES, D), jnp.float32),
            pltpu.VMEM_SHARED((NPAD, D), jnp.float32),
            pltpu.SemaphoreType.DMA,
            pltpu.SemaphoreType.DMA,
            pltpu.SemaphoreType.DMA,
            pltpu.SemaphoreType.DMA,
            pltpu.SemaphoreType.DMA,
        ],
    )
    def sc_aggregate(table_hbm, src_hbm, dst_hbm, out_hbm,
                     sidx, didx, rbufa, rbufb, acc,
                     isem, gsema, gsemb, ssema, ssemb):
        c = lax.axis_index("c")
        s = lax.axis_index("s")
        w = c * 16 + s
        rbuf = (rbufa, rbufb)
        gsem = (gsema, gsemb)
        ssem = (ssema, ssemb)

        def fz(i, _):
            rbufa[i // 8, pl.ds((i % 8) * 16, 16)] = jnp.zeros(
                (16,), jnp.float32
            )
            return 0

        lax.fori_loop(0, LANES * 8, fz, 0)
        for k in range(RPT // LANES):
            pltpu.sync_copy(rbufa, acc.at[pl.ds(s * RPT + k * LANES, LANES)])
        plsc.subcore_barrier()

        def group(i, _):
            d1 = pltpu.async_copy(
                src_hbm.at[pl.ds(w * CH + i * GRP, GRP)], sidx, isem
            )
            d2 = pltpu.async_copy(
                dst_hbm.at[pl.ds(s * CH + i * GRP, GRP)], didx, isem
            )
            d1.wait()
            d2.wait()
            sca = [None, None]
            gcur = pltpu.async_copy(table_hbm.at[sidx.at[0]], rbufa, gsema)
            for j in range(GRP):
                b = j % 2
                nb = (j + 1) % 2
                gcur.wait()
                if j + 1 < GRP:
                    if sca[nb] is not None:
                        sca[nb].wait()
                    gnext = pltpu.async_copy(
                        table_hbm.at[sidx.at[j + 1]], rbuf[nb], gsem[nb]
                    )
                sca[b] = pltpu.async_copy(
                    rbuf[b], acc.at[didx.at[j]], ssem[b], add=True
                )
                if j + 1 < GRP:
                    gcur = gnext
            sca[0].wait()
            sca[1].wait()
            return 0

        lax.fori_loop(0, CH // GRP, group, 0)
        plsc.subcore_barrier()
        for k in range(RPT // LANES):
            pltpu.sync_copy(
                acc.at[pl.ds(s * RPT + k * LANES, LANES)],
                out_hbm.at[pl.ds(c * NPAD + s * RPT + k * LANES, LANES)],
            )

    return sc_degree, sc_aggregate


def _tc1_body(x_ref, w_ref, deg_ref, out_ref):
    xw = jnp.dot(x_ref[...], w_ref[...], preferred_element_type=jnp.float32)
    deg = deg_ref[0] + deg_ref[1] + 1.0
    dinv = lax.rsqrt(deg)[:, 0:1]
    xs = xw * dinv
    out_ref[0] = xs[:, :D]
    out_ref[1] = xs[:, D:]


def _tc2_body(agg_ref, xs_ref, deg_ref, w_ref, b_ref, out_ref):
    a = jnp.concatenate(
        [agg_ref[0] + xs_ref[0], agg_ref[1] + xs_ref[1]], axis=1
    )
    deg = deg_ref[0] + deg_ref[1] + 1.0
    dinv = lax.rsqrt(deg)[:, 0:1]
    h = jnp.maximum(a * dinv + b_ref[...], 0.0)
    hw = jnp.dot(h, w_ref[...], preferred_element_type=jnp.float32)
    hs = hw * dinv
    out_ref[0] = hs[:, :D]
    out_ref[1] = hs[:, D:]


def _tc3_body(agg_ref, xs_ref, deg_ref, b_ref, wfc_ref, bfc_ref, out_ref):
    a = jnp.concatenate(
        [agg_ref[0] + xs_ref[0], agg_ref[1] + xs_ref[1]], axis=1
    )
    deg = deg_ref[0] + deg_ref[1] + 1.0
    dinv = lax.rsqrt(deg)[:, 0:1]
    h = jnp.maximum(a * dinv + b_ref[...], 0.0)
    out_ref[...] = (
        jnp.dot(h, wfc_ref[...], preferred_element_type=jnp.float32)
        + bfc_ref[...]
    )


_GRID = (NPAD // BLK,)


def _row_spec(shape):
    nd = len(shape)
    if nd == 2:
        return pl.BlockSpec((BLK, shape[1]), lambda i: (i, 0))
    return pl.BlockSpec((shape[0], BLK, shape[2]), lambda i: (0, i, 0))


def _full_spec(shape):
    return pl.BlockSpec(shape, lambda i: (0,) * len(shape))


def _tc1(xpad, W1, deg2):
    return pl.pallas_call(
        _tc1_body,
        grid=_GRID,
        in_specs=[
            _row_spec((NPAD, 128)),
            _full_spec((128, 256)),
            _row_spec((2, NPAD, DEGW)),
        ],
        out_specs=_row_spec((2, NPAD, D)),
        out_shape=jax.ShapeDtypeStruct((2, NPAD, D), jnp.float32),
    )(xpad, W1, deg2)


def _tc2(agg1, xw1s, deg2, W2, b1):
    return pl.pallas_call(
        _tc2_body,
        grid=_GRID,
        in_specs=[
            _row_spec((2, NPAD, D)),
            _row_spec((2, NPAD, D)),
            _row_spec((2, NPAD, DEGW)),
            _full_spec((256, 256)),
            _full_spec((1, 256)),
        ],
        out_specs=_row_spec((2, NPAD, D)),
        out_shape=jax.ShapeDtypeStruct((2, NPAD, D), jnp.float32),
    )(agg1, xw1s, deg2, W2, b1)


def _tc3(agg2, h1s, deg2, b2, Wfc, bfc):
    return pl.pallas_call(
        _tc3_body,
        grid=_GRID,
        in_specs=[
            _row_spec((2, NPAD, D)),
            _row_spec((2, NPAD, D)),
            _row_spec((2, NPAD, DEGW)),
            _full_spec((1, 256)),
            _full_spec((256, 128)),
            _full_spec((1, 128)),
        ],
        out_specs=_row_spec((NPAD, 128)),
        out_shape=jax.ShapeDtypeStruct((NPAD, 128), jnp.float32),
    )(agg2, h1s, deg2, b2, Wfc, bfc)


def kernel(x, edge_index, W1, b1, W2, b2, Wfc, bfc):
    src = edge_index[0].astype(jnp.int32)
    dst = edge_index[1].astype(jnp.int32)
    pad = EPAD - E
    srcp = jnp.concatenate([src, jnp.zeros((pad,), jnp.int32)])
    dstp = jnp.concatenate([dst, jnp.full((pad,), N, jnp.int32)])
    src2 = jnp.concatenate([srcp, srcp + NPAD]).reshape(32 * CH, LANES)
    dstf = dstp.reshape(16 * CH, LANES)
    xpad = jnp.pad(x, ((0, NPAD - N), (0, 0)))

    sc_degree, sc_aggregate = _sc_kernels()
    deg2 = sc_degree(dstf).reshape(2, NPAD, DEGW)
    xw1s = _tc1(xpad, W1, deg2)
    agg1 = sc_aggregate(xw1s.reshape(2 * NPAD, D), src2, dstf)
    h1s = _tc2(agg1.reshape(2, NPAD, D), xw1s, deg2, W2, b1.reshape(1, 256))
    agg2 = sc_aggregate(h1s.reshape(2 * NPAD, D), src2, dstf)
    out = _tc3(
        agg2.reshape(2, NPAD, D),
        h1s,
        deg2,
        b2.reshape(1, 256),
        Wfc,
        bfc.reshape(1, 128),
    )
    return out[:N]


# speedup vs baseline: 8.2956x; 1.0206x over previous
import functools

import jax
import jax.numpy as jnp
from jax import lax
from jax.experimental import pallas as pl
from jax.experimental.pallas import tpu as pltpu
from jax.experimental.pallas import tpu_sc as plsc

N = 10000
NPAD = 10240
E = 320000
LANES = 128
CH = 160
GRP = 8
EPT = CH * LANES
EPAD = 16 * EPT
CHD = CH // 2
D = 128
DEGW = 128
DGRP = 8
DNG = CHD // DGRP
RPT = NPAD // 16
BLK = 256

@functools.cache
def _sc_kernels():
    mesh = plsc.VectorSubcoreMesh(
        core_axis_name="c", subcore_axis_name="s", num_cores=2, num_subcores=16
    )

    @functools.partial(
        pl.kernel,
        out_type=jax.ShapeDtypeStruct((2 * NPAD, DEGW), jnp.float32),
        mesh=mesh,
        scratch_types=[
            pltpu.VMEM((DGRP, LANES), jnp.int32),
            pltpu.VMEM((LANES, DEGW), jnp.float32),
            pltpu.VMEM_SHARED((NPAD, DEGW), jnp.float32),
            pltpu.SemaphoreType.DMA,
        ],
    )
    def sc_degree(dst_hbm, out_hbm, didx, ones_v, acc, ssem):
        c = lax.axis_index("c")
        s = lax.axis_index("s")

        def fill_zeros(i, _):
            ones_v[i // 8, pl.ds((i % 8) * 16, 16)] = jnp.zeros(
                (16,), jnp.float32
            )
            return 0

        lax.fori_loop(0, LANES * 8, fill_zeros, 0)
        for k in range(RPT // LANES):
            pltpu.sync_copy(ones_v, acc.at[pl.ds(s * RPT + k * LANES, LANES)])
        plsc.subcore_barrier()

        def fill_ones(i, _):
            ones_v[i // 8, pl.ds((i % 8) * 16, 16)] = jnp.full(
                (16,), 1.0, jnp.float32
            )
            return 0

        lax.fori_loop(0, LANES * 8, fill_ones, 0)

        def group(i, _):
            pltpu.sync_copy(
                dst_hbm.at[pl.ds(s * CH + c * CHD + i * DGRP, DGRP)], didx
            )
            descs = [
                pltpu.async_copy(ones_v, acc.at[didx.at[j]], ssem, add=True)
                for j in range(DGRP)
            ]
            for d in descs:
                d.wait()
            return 0

        lax.fori_loop(0, DNG, group, 0)
        plsc.subcore_barrier()
        for k in range(RPT // LANES):
            pltpu.sync_copy(
                acc.at[pl.ds(s * RPT + k * LANES, LANES)],
                out_hbm.at[pl.ds(c * NPAD + s * RPT + k * LANES, LANES)],
            )

    @functools.partial(
        pl.kernel,
        out_type=jax.ShapeDtypeStruct((2 * NPAD, D), jnp.float32),
        mesh=mesh,
        scratch_types=[
            pltpu.VMEM((GRP, LANES), jnp.int32),
            pltpu.VMEM((GRP, LANES), jnp.int32),
            pltpu.VMEM((LANES, D), jnp.float32),
            pltpu.VMEM((LANES, D), jnp.float32),
            pltpu.VMEM_SHARED((NPAD, D), jnp.float32),
            pltpu.SemaphoreType.DMA,
            pltpu.SemaphoreType.DMA,
            pltpu.SemaphoreType.DMA,
            pltpu.SemaphoreType.DMA,
            pltpu.SemaphoreType.DMA,
        ],
    )
    def sc_aggregate(table_hbm, src_hbm, dst_hbm, out_hbm,
                     sidx, didx, rbufa, rbufb, acc,
                     isem, gsema, gsemb, ssema, ssemb):
        c = lax.axis_index("c")
        s = lax.axis_index("s")
        w = c * 16 + s
        rbuf = (rbufa, rbufb)
        gsem = (gsema, gsemb)
        ssem = (ssema, ssemb)

        def fz(i, _):
            rbufa[i // 8, pl.ds((i % 8) * 16, 16)] = jnp.zeros(
                (16,), jnp.float32
            )
            return 0

        lax.fori_loop(0, LANES * 8, fz, 0)
        for k in range(RPT // LANES):
            pltpu.sync_copy(rbufa, acc.at[pl.ds(s * RPT + k * LANES, LANES)])
        plsc.subcore_barrier()

        def group(i, _):
            d1 = pltpu.async_copy(
                src_hbm.at[pl.ds(w * CH + i * GRP, GRP)], sidx, isem
            )
            d2 = pltpu.async_copy(
                dst_hbm.at[pl.ds(s * CH + i * GRP, GRP)], didx, isem
            )
            d1.wait()
            d2.wait()
            sca = [None, None]
            gcur = pltpu.async_copy(table_hbm.at[sidx.at[0]], rbufa, gsema)
            for j in range(GRP):
                b = j % 2
                nb = (j + 1) % 2
                gcur.wait()
                if j + 1 < GRP:
                    if sca[nb] is not None:
                        sca[nb].wait()
                    gnext = pltpu.async_copy(
                        table_hbm.at[sidx.at[j + 1]], rbuf[nb], gsem[nb]
                    )
                sca[b] = pltpu.async_copy(
                    rbuf[b], acc.at[didx.at[j]], ssem[b], add=True
                )
                if j + 1 < GRP:
                    gcur = gnext
            if sca[0] is not None:
                sca[0].wait()
            if sca[1] is not None:
                sca[1].wait()
            return 0

        lax.fori_loop(0, CH // GRP, group, 0)
        plsc.subcore_barrier()
        for k in range(RPT // LANES):
            pltpu.sync_copy(
                acc.at[pl.ds(s * RPT + k * LANES, LANES)],
                out_hbm.at[pl.ds(c * NPAD + s * RPT + k * LANES, LANES)],
            )

    return sc_degree, sc_aggregate


def _tc1_body(x_ref, w_ref, deg_ref, out_ref):
    xw = jnp.dot(x_ref[...], w_ref[...], preferred_element_type=jnp.float32)
    deg = deg_ref[0] + deg_ref[1] + 1.0
    dinv = lax.rsqrt(deg)[:, 0:1]
    xs = xw * dinv
    out_ref[0] = xs[:, :D]
    out_ref[1] = xs[:, D:]


def _tc2_body(agg_ref, xs_ref, deg_ref, w_ref, b_ref, out_ref):
    a = jnp.concatenate(
        [agg_ref[0] + xs_ref[0], agg_ref[1] + xs_ref[1]], axis=1
    )
    deg = deg_ref[0] + deg_ref[1] + 1.0
    dinv = lax.rsqrt(deg)[:, 0:1]
    h = jnp.maximum(a * dinv + b_ref[...], 0.0)
    hw = jnp.dot(h, w_ref[...], preferred_element_type=jnp.float32)
    hs = hw * dinv
    out_ref[0] = hs[:, :D]
    out_ref[1] = hs[:, D:]


def _tc3_body(agg_ref, xs_ref, deg_ref, b_ref, wfc_ref, bfc_ref, out_ref):
    a = jnp.concatenate(
        [agg_ref[0] + xs_ref[0], agg_ref[1] + xs_ref[1]], axis=1
    )
    deg = deg_ref[0] + deg_ref[1] + 1.0
    dinv = lax.rsqrt(deg)[:, 0:1]
    h = jnp.maximum(a * dinv + b_ref[...], 0.0)
    out_ref[...] = (
        jnp.dot(h, wfc_ref[...], preferred_element_type=jnp.float32)
        + bfc_ref[...]
    )


_GRID = (NPAD // BLK,)


def _row_spec(shape):
    nd = len(shape)
    if nd == 2:
        return pl.BlockSpec((BLK, shape[1]), lambda i: (i, 0))
    return pl.BlockSpec((shape[0], BLK, shape[2]), lambda i: (0, i, 0))


def _full_spec(shape):
    return pl.BlockSpec(shape, lambda i: (0,) * len(shape))


def _tc1(xpad, W1, deg2):
    return pl.pallas_call(
        _tc1_body,
        grid=_GRID,
        in_specs=[
            _row_spec((NPAD, 128)),
            _full_spec((128, 256)),
            _row_spec((2, NPAD, DEGW)),
        ],
        out_specs=_row_spec((2, NPAD, D)),
        out_shape=jax.ShapeDtypeStruct((2, NPAD, D), jnp.float32),
    )(xpad, W1, deg2)


def _tc2(agg1, xw1s, deg2, W2, b1):
    return pl.pallas_call(
        _tc2_body,
        grid=_GRID,
        in_specs=[
            _row_spec((2, NPAD, D)),
            _row_spec((2, NPAD, D)),
            _row_spec((2, NPAD, DEGW)),
            _full_spec((256, 256)),
            _full_spec((1, 256)),
        ],
        out_specs=_row_spec((2, NPAD, D)),
        out_shape=jax.ShapeDtypeStruct((2, NPAD, D), jnp.float32),
    )(agg1, xw1s, deg2, W2, b1)


def _tc3(agg2, h1s, deg2, b2, Wfc, bfc):
    return pl.pallas_call(
        _tc3_body,
        grid=_GRID,
        in_specs=[
            _row_spec((2, NPAD, D)),
            _row_spec((2, NPAD, D)),
            _row_spec((2, NPAD, DEGW)),
            _full_spec((1, 256)),
            _full_spec((256, 128)),
            _full_spec((1, 128)),
        ],
        out_specs=_row_spec((NPAD, 128)),
        out_shape=jax.ShapeDtypeStruct((NPAD, 128), jnp.float32),
    )(agg2, h1s, deg2, b2, Wfc, bfc)


def kernel(x, edge_index, W1, b1, W2, b2, Wfc, bfc):
    src = edge_index[0].astype(jnp.int32)
    dst = edge_index[1].astype(jnp.int32)
    pad = EPAD - E
    srcp = jnp.concatenate([src, jnp.zeros((pad,), jnp.int32)])
    dstp = jnp.concatenate([dst, jnp.full((pad,), N, jnp.int32)])
    src2 = jnp.concatenate([srcp, srcp + NPAD]).reshape(32 * CH, LANES)
    dstf = dstp.reshape(16 * CH, LANES)
    xpad = jnp.pad(x, ((0, NPAD - N), (0, 0)))

    sc_degree, sc_aggregate = _sc_kernels()
    deg2 = sc_degree(dstf).reshape(2, NPAD, DEGW)
    xw1s = _tc1(xpad, W1, deg2)
    agg1 = sc_aggregate(xw1s.reshape(2 * NPAD, D), src2, dstf)
    h1s = _tc2(agg1.reshape(2, NPAD, D), xw1s, deg2, W2, b1.reshape(1, 256))
    agg2 = sc_aggregate(h1s.reshape(2 * NPAD, D), src2, dstf)
    out = _tc3(
        agg2.reshape(2, NPAD, D),
        h1s,
        deg2,
        b2.reshape(1, 256),
        Wfc,
        bfc.reshape(1, 128),
    )
    return out[:N]


# speedup vs baseline: 8.6260x; 1.0398x over previous
import functools

import jax
import jax.numpy as jnp
from jax import lax
from jax.experimental import pallas as pl
from jax.experimental.pallas import tpu as pltpu
from jax.experimental.pallas import tpu_sc as plsc

N = 10000
NPAD = 10240
E = 320000
LANES = 128
CH = 160
GRP = 8
EPT = CH * LANES
EPAD = 16 * EPT
CHD = CH // 2
D = 128
DEGW = 128
DGRP = 8
DNG = CHD // DGRP
RPT = NPAD // 16
BLK = 256

@functools.cache
def _sc_kernels():
    mesh = plsc.VectorSubcoreMesh(
        core_axis_name="c", subcore_axis_name="s", num_cores=2, num_subcores=16
    )

    @functools.partial(
        pl.kernel,
        out_type=jax.ShapeDtypeStruct((2 * NPAD, DEGW), jnp.float32),
        mesh=mesh,
        scratch_types=[
            pltpu.VMEM((DGRP, LANES), jnp.int32),
            pltpu.VMEM((LANES, DEGW), jnp.float32),
            pltpu.VMEM_SHARED((NPAD, DEGW), jnp.float32),
            pltpu.SemaphoreType.DMA,
        ],
    )
    def sc_degree(dst_hbm, out_hbm, didx, ones_v, acc, ssem):
        c = lax.axis_index("c")
        s = lax.axis_index("s")

        def fill_zeros(i, _):
            ones_v[i // 8, pl.ds((i % 8) * 16, 16)] = jnp.zeros(
                (16,), jnp.float32
            )
            return 0

        lax.fori_loop(0, LANES * 8, fill_zeros, 0)
        for k in range(RPT // LANES):
            pltpu.sync_copy(ones_v, acc.at[pl.ds(s * RPT + k * LANES, LANES)])
        plsc.subcore_barrier()

        def fill_ones(i, _):
            ones_v[i // 8, pl.ds((i % 8) * 16, 16)] = jnp.full(
                (16,), 1.0, jnp.float32
            )
            return 0

        lax.fori_loop(0, LANES * 8, fill_ones, 0)

        def group(i, _):
            pltpu.sync_copy(
                dst_hbm.at[pl.ds(s * CH + c * CHD + i * DGRP, DGRP)], didx
            )
            descs = [
                pltpu.async_copy(ones_v, acc.at[didx.at[j]], ssem, add=True)
                for j in range(DGRP)
            ]
            for d in descs:
                d.wait()
            return 0

        lax.fori_loop(0, DNG, group, 0)
        plsc.subcore_barrier()
        for k in range(RPT // LANES):
            pltpu.sync_copy(
                acc.at[pl.ds(s * RPT + k * LANES, LANES)],
                out_hbm.at[pl.ds(c * NPAD + s * RPT + k * LANES, LANES)],
            )

    @functools.partial(
        pl.kernel,
        out_type=jax.ShapeDtypeStruct((2 * NPAD, D), jnp.float32),
        mesh=mesh,
        scratch_types=[
            pltpu.VMEM((GRP, LANES), jnp.int32),
            pltpu.VMEM((GRP, LANES), jnp.int32),
            pltpu.VMEM((LANES, D), jnp.float32),
            pltpu.VMEM((LANES, D), jnp.float32),
            pltpu.VMEM_SHARED((NPAD, D), jnp.float32),
            pltpu.SemaphoreType.DMA,
            pltpu.SemaphoreType.DMA,
            pltpu.SemaphoreType.DMA,
            pltpu.SemaphoreType.DMA,
            pltpu.SemaphoreType.DMA,
        ],
    )
    def sc_aggregate(table_hbm, src_hbm, dst_hbm, out_hbm,
                     sidx, didx, rbufa, rbufb, acc,
                     isem, gsema, gsemb, ssema, ssemb):
        c = lax.axis_index("c")
        s = lax.axis_index("s")
        w = c * 16 + s
        rbuf = (rbufa, rbufb)
        gsem = (gsema, gsemb)
        ssem = (ssema, ssemb)

        def fz(i, _):
            rbufa[i // 8, pl.ds((i % 8) * 16, 16)] = jnp.zeros(
                (16,), jnp.float32
            )
            return 0

        lax.fori_loop(0, LANES * 8, fz, 0)
        for k in range(RPT // LANES):
            pltpu.sync_copy(rbufa, acc.at[pl.ds(s * RPT + k * LANES, LANES)])
        plsc.subcore_barrier()

        def group(i, _):
            d1 = pltpu.async_copy(
                src_hbm.at[pl.ds(w * CH + i * GRP, GRP)], sidx, isem
            )
            d2 = pltpu.async_copy(
                dst_hbm.at[pl.ds(s * CH + i * GRP, GRP)], didx, isem
            )
            d1.wait()
            d2.wait()
            gat = [None] * GRP
            sca = [None] * GRP
            gat[0] = pltpu.async_copy(table_hbm.at[sidx.at[0]], rbufa, gsema)
            gat[1] = pltpu.async_copy(table_hbm.at[sidx.at[1]], rbufb, gsemb)
            for j in range(GRP):
                b = j % 2
                gat[j].wait()
                sca[j] = pltpu.async_copy(
                    rbuf[b], acc.at[didx.at[j]], ssem[b], add=True
                )
                if j + 2 < GRP:
                    sca[j].wait()
                    gat[j + 2] = pltpu.async_copy(
                        table_hbm.at[sidx.at[j + 2]], rbuf[b], gsem[b]
                    )
            sca[GRP - 2].wait()
            sca[GRP - 1].wait()
            return 0

        lax.fori_loop(0, CH // GRP, group, 0)
        plsc.subcore_barrier()
        for k in range(RPT // LANES):
            pltpu.sync_copy(
                acc.at[pl.ds(s * RPT + k * LANES, LANES)],
                out_hbm.at[pl.ds(c * NPAD + s * RPT + k * LANES, LANES)],
            )

    return sc_degree, sc_aggregate


def _tc1_body(x_ref, w_ref, deg_ref, out_ref):
    xw = jnp.dot(x_ref[...], w_ref[...], preferred_element_type=jnp.float32)
    deg = deg_ref[0] + deg_ref[1] + 1.0
    dinv = lax.rsqrt(deg)[:, 0:1]
    xs = xw * dinv
    out_ref[0] = xs[:, :D]
    out_ref[1] = xs[:, D:]


def _tc2_body(agg_ref, xs_ref, deg_ref, w_ref, b_ref, out_ref):
    a = jnp.concatenate(
        [agg_ref[0] + xs_ref[0], agg_ref[1] + xs_ref[1]], axis=1
    )
    deg = deg_ref[0] + deg_ref[1] + 1.0
    dinv = lax.rsqrt(deg)[:, 0:1]
    h = jnp.maximum(a * dinv + b_ref[...], 0.0)
    hw = jnp.dot(h, w_ref[...], preferred_element_type=jnp.float32)
    hs = hw * dinv
    out_ref[0] = hs[:, :D]
    out_ref[1] = hs[:, D:]


def _tc3_body(agg_ref, xs_ref, deg_ref, b_ref, wfc_ref, bfc_ref, out_ref):
    a = jnp.concatenate(
        [agg_ref[0] + xs_ref[0], agg_ref[1] + xs_ref[1]], axis=1
    )
    deg = deg_ref[0] + deg_ref[1] + 1.0
    dinv = lax.rsqrt(deg)[:, 0:1]
    h = jnp.maximum(a * dinv + b_ref[...], 0.0)
    out_ref[...] = (
        jnp.dot(h, wfc_ref[...], preferred_element_type=jnp.float32)
        + bfc_ref[...]
    )


_GRID = (NPAD // BLK,)


def _row_spec(shape):
    nd = len(shape)
    if nd == 2:
        return pl.BlockSpec((BLK, shape[1]), lambda i: (i, 0))
    return pl.BlockSpec((shape[0], BLK, shape[2]), lambda i: (0, i, 0))


def _full_spec(shape):
    return pl.BlockSpec(shape, lambda i: (0,) * len(shape))


def _tc1(xpad, W1, deg2):
    return pl.pallas_call(
        _tc1_body,
        grid=_GRID,
        in_specs=[
            _row_spec((NPAD, 128)),
            _full_spec((128, 256)),
            _row_spec((2, NPAD, DEGW)),
        ],
        out_specs=_row_spec((2, NPAD, D)),
        out_shape=jax.ShapeDtypeStruct((2, NPAD, D), jnp.float32),
    )(xpad, W1, deg2)


def _tc2(agg1, xw1s, deg2, W2, b1):
    return pl.pallas_call(
        _tc2_body,
        grid=_GRID,
        in_specs=[
            _row_spec((2, NPAD, D)),
            _row_spec((2, NPAD, D)),
            _row_spec((2, NPAD, DEGW)),
            _full_spec((256, 256)),
            _full_spec((1, 256)),
        ],
        out_specs=_row_spec((2, NPAD, D)),
        out_shape=jax.ShapeDtypeStruct((2, NPAD, D), jnp.float32),
    )(agg1, xw1s, deg2, W2, b1)


def _tc3(agg2, h1s, deg2, b2, Wfc, bfc):
    return pl.pallas_call(
        _tc3_body,
        grid=_GRID,
        in_specs=[
            _row_spec((2, NPAD, D)),
            _row_spec((2, NPAD, D)),
            _row_spec((2, NPAD, DEGW)),
            _full_spec((1, 256)),
            _full_spec((256, 128)),
            _full_spec((1, 128)),
        ],
        out_specs=_row_spec((NPAD, 128)),
        out_shape=jax.ShapeDtypeStruct((NPAD, 128), jnp.float32),
    )(agg2, h1s, deg2, b2, Wfc, bfc)


def kernel(x, edge_index, W1, b1, W2, b2, Wfc, bfc):
    src = edge_index[0].astype(jnp.int32)
    dst = edge_index[1].astype(jnp.int32)
    pad = EPAD - E
    srcp = jnp.concatenate([src, jnp.zeros((pad,), jnp.int32)])
    dstp = jnp.concatenate([dst, jnp.full((pad,), N, jnp.int32)])
    src2 = jnp.concatenate([srcp, srcp + NPAD]).reshape(32 * CH, LANES)
    dstf = dstp.reshape(16 * CH, LANES)
    xpad = jnp.pad(x, ((0, NPAD - N), (0, 0)))

    sc_degree, sc_aggregate = _sc_kernels()
    deg2 = sc_degree(dstf).reshape(2, NPAD, DEGW)
    xw1s = _tc1(xpad, W1, deg2)
    agg1 = sc_aggregate(xw1s.reshape(2 * NPAD, D), src2, dstf)
    h1s = _tc2(agg1.reshape(2, NPAD, D), xw1s, deg2, W2, b1.reshape(1, 256))
    agg2 = sc_aggregate(h1s.reshape(2 * NPAD, D), src2, dstf)
    out = _tc3(
        agg2.reshape(2, NPAD, D),
        h1s,
        deg2,
        b2.reshape(1, 256),
        Wfc,
        bfc.reshape(1, 128),
    )
    return out[:N]


# speedup vs baseline: 8.6447x; 1.0022x over previous
import functools

import jax
import jax.numpy as jnp
from jax import lax
from jax.experimental import pallas as pl
from jax.experimental.pallas import tpu as pltpu
from jax.experimental.pallas import tpu_sc as plsc

N = 10000
NPAD = 10240
E = 320000
LANES = 128
CH = 160
GRP = 8
EPT = CH * LANES
EPAD = 16 * EPT
CHD = CH // 2
D = 128
DEGW = 128
DGRP = 8
DNG = CHD // DGRP
RPT = NPAD // 16
BLK = 256

@functools.cache
def _sc_kernels():
    mesh = plsc.VectorSubcoreMesh(
        core_axis_name="c", subcore_axis_name="s", num_cores=2, num_subcores=16
    )

    @functools.partial(
        pl.kernel,
        out_type=jax.ShapeDtypeStruct((2 * NPAD, DEGW), jnp.float32),
        mesh=mesh,
        scratch_types=[
            pltpu.VMEM((DGRP, LANES), jnp.int32),
            pltpu.VMEM((LANES, DEGW), jnp.float32),
            pltpu.VMEM_SHARED((NPAD, DEGW), jnp.float32),
            pltpu.SemaphoreType.DMA,
        ],
    )
    def sc_degree(dst_hbm, out_hbm, didx, ones_v, acc, ssem):
        c = lax.axis_index("c")
        s = lax.axis_index("s")

        def fill_zeros(i, _):
            ones_v[i // 8, pl.ds((i % 8) * 16, 16)] = jnp.zeros(
                (16,), jnp.float32
            )
            return 0

        lax.fori_loop(0, LANES * 8, fill_zeros, 0)
        for k in range(RPT // LANES):
            pltpu.sync_copy(ones_v, acc.at[pl.ds(s * RPT + k * LANES, LANES)])
        plsc.subcore_barrier()

        def fill_ones(i, _):
            ones_v[i // 8, pl.ds((i % 8) * 16, 16)] = jnp.full(
                (16,), 1.0, jnp.float32
            )
            return 0

        lax.fori_loop(0, LANES * 8, fill_ones, 0)

        def group(i, _):
            pltpu.sync_copy(
                dst_hbm.at[pl.ds(s * CH + c * CHD + i * DGRP, DGRP)], didx
            )
            descs = [
                pltpu.async_copy(ones_v, acc.at[didx.at[j]], ssem, add=True)
                for j in range(DGRP)
            ]
            for d in descs:
                d.wait()
            return 0

        lax.fori_loop(0, DNG, group, 0)
        plsc.subcore_barrier()
        for k in range(RPT // LANES):
            pltpu.sync_copy(
                acc.at[pl.ds(s * RPT + k * LANES, LANES)],
                out_hbm.at[pl.ds(c * NPAD + s * RPT + k * LANES, LANES)],
            )

    @functools.partial(
        pl.kernel,
        out_type=jax.ShapeDtypeStruct((2 * NPAD, D), jnp.float32),
        mesh=mesh,
        scratch_types=[
            pltpu.VMEM((GRP, LANES), jnp.int32),
            pltpu.VMEM((GRP, LANES), jnp.int32),
            pltpu.VMEM((LANES, D), jnp.float32),
            pltpu.VMEM((LANES, D), jnp.float32),
            pltpu.VMEM_SHARED((NPAD, D), jnp.float32),
            pltpu.SemaphoreType.DMA,
            pltpu.SemaphoreType.DMA,
            pltpu.SemaphoreType.DMA,
            pltpu.SemaphoreType.DMA,
            pltpu.SemaphoreType.DMA,
        ],
    )
    def sc_aggregate(table_hbm, src_hbm, dst_hbm, out_hbm,
                     sidx, didx, rbufa, rbufb, acc,
                     isem, gsema, gsemb, ssema, ssemb):
        c = lax.axis_index("c")
        s = lax.axis_index("s")
        w = c * 16 + s
        rbuf = (rbufa, rbufb)
        gsem = (gsema, gsemb)
        ssem = (ssema, ssemb)

        def fz(i, _):
            rbufa[i // 8, pl.ds((i % 8) * 16, 16)] = jnp.zeros(
                (16,), jnp.float32
            )
            return 0

        lax.fori_loop(0, LANES * 8, fz, 0)
        for k in range(RPT // LANES):
            pltpu.sync_copy(rbufa, acc.at[pl.ds(s * RPT + k * LANES, LANES)])
        plsc.subcore_barrier()

        ng = CH // GRP
        pltpu.async_copy(src_hbm.at[pl.ds(w * CH, GRP)], sidx, isem)
        pltpu.async_copy(dst_hbm.at[pl.ds(s * CH, GRP)], didx, isem)

        def group(i, _):
            pltpu.make_async_copy(
                src_hbm.at[pl.ds(0, GRP)], sidx, isem
            ).wait()
            pltpu.make_async_copy(
                dst_hbm.at[pl.ds(0, GRP)], didx, isem
            ).wait()
            nxt = lax.rem(i + 1, ng)
            gat = [None] * GRP
            sca = [None] * GRP
            gat[0] = pltpu.async_copy(table_hbm.at[sidx.at[0]], rbufa, gsema)
            gat[1] = pltpu.async_copy(table_hbm.at[sidx.at[1]], rbufb, gsemb)
            for j in range(GRP):
                b = j % 2
                gat[j].wait()
                if j == GRP - 1:
                    pltpu.async_copy(
                        src_hbm.at[pl.ds(w * CH + nxt * GRP, GRP)], sidx, isem
                    )
                sca[j] = pltpu.async_copy(
                    rbuf[b], acc.at[didx.at[j]], ssem[b], add=True
                )
                if j + 2 < GRP:
                    sca[j].wait()
                    gat[j + 2] = pltpu.async_copy(
                        table_hbm.at[sidx.at[j + 2]], rbuf[b], gsem[b]
                    )
            sca[GRP - 2].wait()
            sca[GRP - 1].wait()
            pltpu.async_copy(
                dst_hbm.at[pl.ds(s * CH + nxt * GRP, GRP)], didx, isem
            )
            return 0

        lax.fori_loop(0, ng, group, 0)
        pltpu.make_async_copy(src_hbm.at[pl.ds(0, GRP)], sidx, isem).wait()
        pltpu.make_async_copy(dst_hbm.at[pl.ds(0, GRP)], didx, isem).wait()
        plsc.subcore_barrier()
        for k in range(RPT // LANES):
            pltpu.sync_copy(
                acc.at[pl.ds(s * RPT + k * LANES, LANES)],
                out_hbm.at[pl.ds(c * NPAD + s * RPT + k * LANES, LANES)],
            )

    return sc_degree, sc_aggregate


def _tc1_body(x_ref, w_ref, deg_ref, out_ref):
    xw = jnp.dot(x_ref[...], w_ref[...], preferred_element_type=jnp.float32)
    deg = deg_ref[0] + deg_ref[1] + 1.0
    dinv = lax.rsqrt(deg)[:, 0:1]
    xs = xw * dinv
    out_ref[0] = xs[:, :D]
    out_ref[1] = xs[:, D:]


def _tc2_body(agg_ref, xs_ref, deg_ref, w_ref, b_ref, out_ref):
    a = jnp.concatenate(
        [agg_ref[0] + xs_ref[0], agg_ref[1] + xs_ref[1]], axis=1
    )
    deg = deg_ref[0] + deg_ref[1] + 1.0
    dinv = lax.rsqrt(deg)[:, 0:1]
    h = jnp.maximum(a * dinv + b_ref[...], 0.0)
    hw = jnp.dot(h, w_ref[...], preferred_element_type=jnp.float32)
    hs = hw * dinv
    out_ref[0] = hs[:, :D]
    out_ref[1] = hs[:, D:]


def _tc3_body(agg_ref, xs_ref, deg_ref, b_ref, wfc_ref, bfc_ref, out_ref):
    a = jnp.concatenate(
        [agg_ref[0] + xs_ref[0], agg_ref[1] + xs_ref[1]], axis=1
    )
    deg = deg_ref[0] + deg_ref[1] + 1.0
    dinv = lax.rsqrt(deg)[:, 0:1]
    h = jnp.maximum(a * dinv + b_ref[...], 0.0)
    out_ref[...] = (
        jnp.dot(h, wfc_ref[...], preferred_element_type=jnp.float32)
        + bfc_ref[...]
    )


_GRID = (NPAD // BLK,)


def _row_spec(shape):
    nd = len(shape)
    if nd == 2:
        return pl.BlockSpec((BLK, shape[1]), lambda i: (i, 0))
    return pl.BlockSpec((shape[0], BLK, shape[2]), lambda i: (0, i, 0))


def _full_spec(shape):
    return pl.BlockSpec(shape, lambda i: (0,) * len(shape))


def _tc1(xpad, W1, deg2):
    return pl.pallas_call(
        _tc1_body,
        grid=_GRID,
        in_specs=[
            _row_spec((NPAD, 128)),
            _full_spec((128, 256)),
            _row_spec((2, NPAD, DEGW)),
        ],
        out_specs=_row_spec((2, NPAD, D)),
        out_shape=jax.ShapeDtypeStruct((2, NPAD, D), jnp.float32),
    )(xpad, W1, deg2)


def _tc2(agg1, xw1s, deg2, W2, b1):
    return pl.pallas_call(
        _tc2_body,
        grid=_GRID,
        in_specs=[
            _row_spec((2, NPAD, D)),
            _row_spec((2, NPAD, D)),
            _row_spec((2, NPAD, DEGW)),
            _full_spec((256, 256)),
            _full_spec((1, 256)),
        ],
        out_specs=_row_spec((2, NPAD, D)),
        out_shape=jax.ShapeDtypeStruct((2, NPAD, D), jnp.float32),
    )(agg1, xw1s, deg2, W2, b1)


def _tc3(agg2, h1s, deg2, b2, Wfc, bfc):
    return pl.pallas_call(
        _tc3_body,
        grid=_GRID,
        in_specs=[
            _row_spec((2, NPAD, D)),
            _row_spec((2, NPAD, D)),
            _row_spec((2, NPAD, DEGW)),
            _full_spec((1, 256)),
            _full_spec((256, 128)),
            _full_spec((1, 128)),
        ],
        out_specs=_row_spec((NPAD, 128)),
        out_shape=jax.ShapeDtypeStruct((NPAD, 128), jnp.float32),
    )(agg2, h1s, deg2, b2, Wfc, bfc)


def kernel(x, edge_index, W1, b1, W2, b2, Wfc, bfc):
    src = edge_index[0].astype(jnp.int32)
    dst = edge_index[1].astype(jnp.int32)
    pad = EPAD - E
    srcp = jnp.concatenate([src, jnp.zeros((pad,), jnp.int32)])
    dstp = jnp.concatenate([dst, jnp.full((pad,), N, jnp.int32)])
    src2 = jnp.concatenate([srcp, srcp + NPAD]).reshape(32 * CH, LANES)
    dstf = dstp.reshape(16 * CH, LANES)
    xpad = jnp.pad(x, ((0, NPAD - N), (0, 0)))

    sc_degree, sc_aggregate = _sc_kernels()
    deg2 = sc_degree(dstf).reshape(2, NPAD, DEGW)
    xw1s = _tc1(xpad, W1, deg2)
    agg1 = sc_aggregate(xw1s.reshape(2 * NPAD, D), src2, dstf)
    h1s = _tc2(agg1.reshape(2, NPAD, D), xw1s, deg2, W2, b1.reshape(1, 256))
    agg2 = sc_aggregate(h1s.reshape(2 * NPAD, D), src2, dstf)
    out = _tc3(
        agg2.reshape(2, NPAD, D),
        h1s,
        deg2,
        b2.reshape(1, 256),
        Wfc,
        bfc.reshape(1, 128),
    )
    return out[:N]


# speedup vs baseline: 8.8823x; 1.0275x over previous
import functools

import jax
import jax.numpy as jnp
from jax import lax
from jax.experimental import pallas as pl
from jax.experimental.pallas import tpu as pltpu
from jax.experimental.pallas import tpu_sc as plsc

N = 10000
NPAD = 10240
E = 320000
LANES = 128
CH = 160
GRP = 16
EPT = CH * LANES
EPAD = 16 * EPT
CHD = CH // 2
D = 128
DEGW = 128
DGRP = 8
DNG = CHD // DGRP
RPT = NPAD // 16
BLK = 256

@functools.cache
def _sc_kernels():
    mesh = plsc.VectorSubcoreMesh(
        core_axis_name="c", subcore_axis_name="s", num_cores=2, num_subcores=16
    )

    @functools.partial(
        pl.kernel,
        out_type=jax.ShapeDtypeStruct((2 * NPAD, DEGW), jnp.float32),
        mesh=mesh,
        scratch_types=[
            pltpu.VMEM((DGRP, LANES), jnp.int32),
            pltpu.VMEM((LANES, DEGW), jnp.float32),
            pltpu.VMEM_SHARED((NPAD, DEGW), jnp.float32),
            pltpu.SemaphoreType.DMA,
        ],
    )
    def sc_degree(dst_hbm, out_hbm, didx, ones_v, acc, ssem):
        c = lax.axis_index("c")
        s = lax.axis_index("s")

        def fill_zeros(i, _):
            ones_v[i // 8, pl.ds((i % 8) * 16, 16)] = jnp.zeros(
                (16,), jnp.float32
            )
            return 0

        lax.fori_loop(0, LANES * 8, fill_zeros, 0)
        for k in range(RPT // LANES):
            pltpu.sync_copy(ones_v, acc.at[pl.ds(s * RPT + k * LANES, LANES)])
        plsc.subcore_barrier()

        def fill_ones(i, _):
            ones_v[i // 8, pl.ds((i % 8) * 16, 16)] = jnp.full(
                (16,), 1.0, jnp.float32
            )
            return 0

        lax.fori_loop(0, LANES * 8, fill_ones, 0)

        def group(i, _):
            pltpu.sync_copy(
                dst_hbm.at[pl.ds(s * CH + c * CHD + i * DGRP, DGRP)], didx
            )
            descs = [
                pltpu.async_copy(ones_v, acc.at[didx.at[j]], ssem, add=True)
                for j in range(DGRP)
            ]
            for d in descs:
                d.wait()
            return 0

        lax.fori_loop(0, DNG, group, 0)
        plsc.subcore_barrier()
        for k in range(RPT // LANES):
            pltpu.sync_copy(
                acc.at[pl.ds(s * RPT + k * LANES, LANES)],
                out_hbm.at[pl.ds(c * NPAD + s * RPT + k * LANES, LANES)],
            )

    @functools.partial(
        pl.kernel,
        out_type=jax.ShapeDtypeStruct((2 * NPAD, D), jnp.float32),
        mesh=mesh,
        scratch_types=[
            pltpu.VMEM((GRP, LANES), jnp.int32),
            pltpu.VMEM((GRP, LANES), jnp.int32),
            pltpu.VMEM((LANES, D), jnp.float32),
            pltpu.VMEM((LANES, D), jnp.float32),
            pltpu.VMEM_SHARED((NPAD, D), jnp.float32),
            pltpu.SemaphoreType.DMA,
            pltpu.SemaphoreType.DMA,
            pltpu.SemaphoreType.DMA,
            pltpu.SemaphoreType.DMA,
            pltpu.SemaphoreType.DMA,
        ],
    )
    def sc_aggregate(table_hbm, src_hbm, dst_hbm, out_hbm,
                     sidx, didx, rbufa, rbufb, acc,
                     isem, gsema, gsemb, ssema, ssemb):
        c = lax.axis_index("c")
        s = lax.axis_index("s")
        w = c * 16 + s
        rbuf = (rbufa, rbufb)
        gsem = (gsema, gsemb)
        ssem = (ssema, ssemb)

        def fz(i, _):
            rbufa[i // 8, pl.ds((i % 8) * 16, 16)] = jnp.zeros(
                (16,), jnp.float32
            )
            return 0

        lax.fori_loop(0, LANES * 8, fz, 0)
        for k in range(RPT // LANES):
            pltpu.sync_copy(rbufa, acc.at[pl.ds(s * RPT + k * LANES, LANES)])
        plsc.subcore_barrier()

        ng = CH // GRP
        pltpu.async_copy(src_hbm.at[pl.ds(w * CH, GRP)], sidx, isem)
        pltpu.async_copy(dst_hbm.at[pl.ds(s * CH, GRP)], didx, isem)

        def group(i, _):
            pltpu.make_async_copy(
                src_hbm.at[pl.ds(0, GRP)], sidx, isem
            ).wait()
            pltpu.make_async_copy(
                dst_hbm.at[pl.ds(0, GRP)], didx, isem
            ).wait()
            nxt = lax.rem(i + 1, ng)
            gat = [None] * GRP
            sca = [None] * GRP
            gat[0] = pltpu.async_copy(table_hbm.at[sidx.at[0]], rbufa, gsema)
            gat[1] = pltpu.async_copy(table_hbm.at[sidx.at[1]], rbufb, gsemb)
            for j in range(GRP):
                b = j % 2
                gat[j].wait()
                if j == GRP - 1:
                    pltpu.async_copy(
                        src_hbm.at[pl.ds(w * CH + nxt * GRP, GRP)], sidx, isem
                    )
                sca[j] = pltpu.async_copy(
                    rbuf[b], acc.at[didx.at[j]], ssem[b], add=True
                )
                if j + 2 < GRP:
                    sca[j].wait()
                    gat[j + 2] = pltpu.async_copy(
                        table_hbm.at[sidx.at[j + 2]], rbuf[b], gsem[b]
                    )
            sca[GRP - 2].wait()
            sca[GRP - 1].wait()
            pltpu.async_copy(
                dst_hbm.at[pl.ds(s * CH + nxt * GRP, GRP)], didx, isem
            )
            return 0

        lax.fori_loop(0, ng, group, 0)
        pltpu.make_async_copy(src_hbm.at[pl.ds(0, GRP)], sidx, isem).wait()
        pltpu.make_async_copy(dst_hbm.at[pl.ds(0, GRP)], didx, isem).wait()
        plsc.subcore_barrier()
        for k in range(RPT // LANES):
            pltpu.sync_copy(
                acc.at[pl.ds(s * RPT + k * LANES, LANES)],
                out_hbm.at[pl.ds(c * NPAD + s * RPT + k * LANES, LANES)],
            )

    return sc_degree, sc_aggregate


def _tc1_body(x_ref, w_ref, deg_ref, out_ref):
    xw = jnp.dot(x_ref[...], w_ref[...], preferred_element_type=jnp.float32)
    deg = deg_ref[0] + deg_ref[1] + 1.0
    dinv = lax.rsqrt(deg)[:, 0:1]
    xs = xw * dinv
    out_ref[0] = xs[:, :D]
    out_ref[1] = xs[:, D:]


def _tc2_body(agg_ref, xs_ref, deg_ref, w_ref, b_ref, out_ref):
    a = jnp.concatenate(
        [agg_ref[0] + xs_ref[0], agg_ref[1] + xs_ref[1]], axis=1
    )
    deg = deg_ref[0] + deg_ref[1] + 1.0
    dinv = lax.rsqrt(deg)[:, 0:1]
    h = jnp.maximum(a * dinv + b_ref[...], 0.0)
    hw = jnp.dot(h, w_ref[...], preferred_element_type=jnp.float32)
    hs = hw * dinv
    out_ref[0] = hs[:, :D]
    out_ref[1] = hs[:, D:]


def _tc3_body(agg_ref, xs_ref, deg_ref, b_ref, wfc_ref, bfc_ref, out_ref):
    a = jnp.concatenate(
        [agg_ref[0] + xs_ref[0], agg_ref[1] + xs_ref[1]], axis=1
    )
    deg = deg_ref[0] + deg_ref[1] + 1.0
    dinv = lax.rsqrt(deg)[:, 0:1]
    h = jnp.maximum(a * dinv + b_ref[...], 0.0)
    out_ref[...] = (
        jnp.dot(h, wfc_ref[...], preferred_element_type=jnp.float32)
        + bfc_ref[...]
    )


_GRID = (NPAD // BLK,)


def _row_spec(shape):
    nd = len(shape)
    if nd == 2:
        return pl.BlockSpec((BLK, shape[1]), lambda i: (i, 0))
    return pl.BlockSpec((shape[0], BLK, shape[2]), lambda i: (0, i, 0))


def _full_spec(shape):
    return pl.BlockSpec(shape, lambda i: (0,) * len(shape))


def _tc1(xpad, W1, deg2):
    return pl.pallas_call(
        _tc1_body,
        grid=_GRID,
        in_specs=[
            _row_spec((NPAD, 128)),
            _full_spec((128, 256)),
            _row_spec((2, NPAD, DEGW)),
        ],
        out_specs=_row_spec((2, NPAD, D)),
        out_shape=jax.ShapeDtypeStruct((2, NPAD, D), jnp.float32),
    )(xpad, W1, deg2)


def _tc2(agg1, xw1s, deg2, W2, b1):
    return pl.pallas_call(
        _tc2_body,
        grid=_GRID,
        in_specs=[
            _row_spec((2, NPAD, D)),
            _row_spec((2, NPAD, D)),
            _row_spec((2, NPAD, DEGW)),
            _full_spec((256, 256)),
            _full_spec((1, 256)),
        ],
        out_specs=_row_spec((2, NPAD, D)),
        out_shape=jax.ShapeDtypeStruct((2, NPAD, D), jnp.float32),
    )(agg1, xw1s, deg2, W2, b1)


def _tc3(agg2, h1s, deg2, b2, Wfc, bfc):
    return pl.pallas_call(
        _tc3_body,
        grid=_GRID,
        in_specs=[
            _row_spec((2, NPAD, D)),
            _row_spec((2, NPAD, D)),
            _row_spec((2, NPAD, DEGW)),
            _full_spec((1, 256)),
            _full_spec((256, 128)),
            _full_spec((1, 128)),
        ],
        out_specs=_row_spec((NPAD, 128)),
        out_shape=jax.ShapeDtypeStruct((NPAD, 128), jnp.float32),
    )(agg2, h1s, deg2, b2, Wfc, bfc)


def kernel(x, edge_index, W1, b1, W2, b2, Wfc, bfc):
    src = edge_index[0].astype(jnp.int32)
    dst = edge_index[1].astype(jnp.int32)
    pad = EPAD - E
    srcp = jnp.concatenate([src, jnp.zeros((pad,), jnp.int32)])
    dstp = jnp.concatenate([dst, jnp.full((pad,), N, jnp.int32)])
    src2 = jnp.concatenate([srcp, srcp + NPAD]).reshape(32 * CH, LANES)
    dstf = dstp.reshape(16 * CH, LANES)
    xpad = jnp.pad(x, ((0, NPAD - N), (0, 0)))

    sc_degree, sc_aggregate = _sc_kernels()
    deg2 = sc_degree(dstf).reshape(2, NPAD, DEGW)
    xw1s = _tc1(xpad, W1, deg2)
    agg1 = sc_aggregate(xw1s.reshape(2 * NPAD, D), src2, dstf)
    h1s = _tc2(agg1.reshape(2, NPAD, D), xw1s, deg2, W2, b1.reshape(1, 256))
    agg2 = sc_aggregate(h1s.reshape(2 * NPAD, D), src2, dstf)
    out = _tc3(
        agg2.reshape(2, NPAD, D),
        h1s,
        deg2,
        b2.reshape(1, 256),
        Wfc,
        bfc.reshape(1, 128),
    )
    return out[:N]
